# Initial kernel scaffold; baseline (speedup 1.0000x reference)
import jax
import jax.numpy as jnp
from jax.experimental import pallas as pl


def kernel(encoded_paths, contexts_per_label, W_attn, W_lin, b_lin):
    raise NotImplementedError("write your pallas kernel here")


# speedup vs baseline: 3.7480x; 3.7480x over previous
import functools

import jax
import jax.numpy as jnp
from jax import lax
from jax.experimental import pallas as pl
from jax.experimental.pallas import tpu as pltpu
from jax.experimental.pallas import tpu_sc as plsc

_N = 32768
_D = 512
_B = 16
_C = 104
_NW = 32
_CHUNK = _N // _NW
_ROWS = 64
_NCH = _CHUNK // _ROWS
_NJ = _D // 16
_NEG = -1.0e30

_HIGH = jax.lax.Precision.HIGHEST


def _wid():
    info = plsc.get_sparse_core_info()
    return lax.axis_index("s") * info.num_cores + lax.axis_index("c")


def _len0_scalar(schedv):
    return schedv[...][0]


def _make_k1():
    mesh = plsc.VectorSubcoreMesh(core_axis_name="c", subcore_axis_name="s")

    @functools.partial(
        pl.kernel,
        mesh=mesh,
        out_type=jax.ShapeDtypeStruct((_NW, 2, _D), jnp.float32),
        scratch_types=[
            pltpu.VMEM((2, _ROWS, _D), jnp.float32),
            pltpu.VMEM((2, _D), jnp.float32),
            pltpu.VMEM((16,), jnp.int32),
            pltpu.SemaphoreType.DMA,
            pltpu.SemaphoreType.DMA,
        ],
    )
    def k1(x_hbm, sched_hbm, out_hbm, xbuf, acc, schedv, sem0, sem1):
        wid = _wid()
        base = wid * _CHUNK
        pltpu.sync_copy(sched_hbm.at[wid], schedv)
        len0 = _len0_scalar(schedv)

        zeros = jnp.zeros((16,), jnp.float32)
        for run in range(2):
            for j in range(_NJ):
                acc[run, pl.ds(16 * j, 16)] = zeros

        sems = (sem0, sem1)

        def dma(c):
            return pltpu.make_async_copy(
                x_hbm.at[pl.ds(base + c * _ROWS, _ROWS), :],
                xbuf.at[c % 2],
                sems[c % 2],
            )

        dma(0).start()
        for c in range(_NCH):
            if c + 1 < _NCH:
                dma(c + 1).start()
            dma(c).wait()
            buf = c % 2
            s_split = jnp.clip(len0 - c * _ROWS, 0, _ROWS)
            for run, lo, hi in ((0, 0, s_split), (1, s_split, _ROWS)):
                init = tuple(acc[run, pl.ds(16 * j, 16)] for j in range(_NJ))

                def body(r, carry, buf=buf):
                    return tuple(
                        carry[j] + xbuf[buf, r, pl.ds(16 * j, 16)]
                        for j in range(_NJ)
                    )

                res = lax.fori_loop(lo, hi, body, init)
                for j in range(_NJ):
                    acc[run, pl.ds(16 * j, 16)] = res[j]

        pltpu.sync_copy(acc, out_hbm.at[wid])

    return k1


def _make_k2():
    mesh = plsc.VectorSubcoreMesh(core_axis_name="c", subcore_axis_name="s")

    @functools.partial(
        pl.kernel,
        mesh=mesh,
        out_type=(
            jax.ShapeDtypeStruct((_NW, 2, 16), jnp.float32),
            jax.ShapeDtypeStruct((_NW, 2, 16), jnp.float32),
            jax.ShapeDtypeStruct((_NW, 2, _D), jnp.float32),
        ),
        scratch_types=[
            pltpu.VMEM((2, _ROWS, _D), jnp.float32),
            pltpu.VMEM((2, _ROWS, 16), jnp.float32),
            pltpu.VMEM((2, _D), jnp.float32),
            pltpu.VMEM((2, 16), jnp.float32),
            pltpu.VMEM((2, 16), jnp.float32),
            pltpu.VMEM((16,), jnp.int32),
            pltpu.SemaphoreType.DMA,
            pltpu.SemaphoreType.DMA,
            pltpu.SemaphoreType.DMA,
            pltpu.SemaphoreType.DMA,
        ],
    )
    def k2(x_hbm, s_hbm, sched_hbm, m_hbm, d_hbm, c_hbm,
           xbuf, svbuf, cacc, dacc, mv, schedv, sem0, sem1, sem2, sem3):
        wid = _wid()
        base = wid * _CHUNK
        pltpu.sync_copy(sched_hbm.at[wid], schedv)
        len0 = _len0_scalar(schedv)

        xsems = (sem0, sem1)
        ssems = (sem2, sem3)

        def xdma(c):
            return pltpu.make_async_copy(
                x_hbm.at[pl.ds(base + c * _ROWS, _ROWS), :],
                xbuf.at[c % 2],
                xsems[c % 2],
            )

        def sdma(c):
            return pltpu.make_async_copy(
                s_hbm.at[pl.ds(base + c * _ROWS, _ROWS), :],
                svbuf.at[c % 2],
                ssems[c % 2],
            )

        neg = jnp.full((16,), _NEG, jnp.float32)
        m_a, m_b = neg, neg
        sdma(0).start()
        for c in range(_NCH):
            if c + 1 < _NCH:
                sdma(c + 1).start()
            sdma(c).wait()
            buf = c % 2
            s_split = jnp.clip(len0 - c * _ROWS, 0, _ROWS)
            m_a = lax.fori_loop(
                0, s_split,
                lambda r, m, buf=buf: jnp.maximum(m, svbuf[buf, r]), m_a)
            m_b = lax.fori_loop(
                s_split, _ROWS,
                lambda r, m, buf=buf: jnp.maximum(m, svbuf[buf, r]), m_b)
        mv[0] = m_a
        mv[1] = m_b

        zeros = jnp.zeros((16,), jnp.float32)
        for run in range(2):
            dacc[run] = zeros
            for j in range(_NJ):
                cacc[run, pl.ds(16 * j, 16)] = zeros

        xdma(0).start()
        sdma(0).start()
        for c in range(_NCH):
            if c + 1 < _NCH:
                xdma(c + 1).start()
                sdma(c + 1).start()
            xdma(c).wait()
            sdma(c).wait()
            buf = c % 2
            s_split = jnp.clip(len0 - c * _ROWS, 0, _ROWS)
            for run, lo, hi, m in ((0, 0, s_split, m_a),
                                   (1, s_split, _ROWS, m_b)):
                init = (dacc[run],) + tuple(
                    cacc[run, pl.ds(16 * j, 16)] for j in range(_NJ))

                def body(r, carry, buf=buf, m=m):
                    d = carry[0]
                    cs = carry[1:]
                    e = jnp.exp(svbuf[buf, r] - m)
                    new_cs = tuple(
                        cs[j] + e * xbuf[buf, r, pl.ds(16 * j, 16)]
                        for j in range(_NJ)
                    )
                    return (d + e,) + new_cs

                res = lax.fori_loop(lo, hi, body, init)
                dacc[run] = res[0]
                for j in range(_NJ):
                    cacc[run, pl.ds(16 * j, 16)] = res[j + 1]

        pltpu.sync_copy(mv, m_hbm.at[wid])
        pltpu.sync_copy(dacc, d_hbm.at[wid])
        pltpu.sync_copy(cacc, c_hbm.at[wid])

    return k2


def _tc_prep(p64, r_mean, w_attn):
    def body(p_ref, r_ref, w_ref, o_ref):
        init = jnp.dot(r_ref[...], p_ref[...], precision=_HIGH,
                       preferred_element_type=jnp.float32)
        o_ref[...] = lax.dot_general(
            init, w_ref[...], (((1,), (1,)), ((), ())), precision=_HIGH,
            preferred_element_type=jnp.float32)

    return pl.pallas_call(
        body, out_shape=jax.ShapeDtypeStruct((_B, _D), jnp.float32),
    )(p64, r_mean, w_attn)


def _tc_score(x, hidden, meta):
    rb = 1024
    grid = _N // rb

    def body(x_ref, h_ref, meta_ref, o_ref):
        i = pl.program_id(0)
        s = lax.dot_general(x_ref[...], h_ref[...], (((1,), (1,)), ((), ())),
                            precision=_HIGH, preferred_element_type=jnp.float32)
        starts = meta_ref[0:1, 0:_B]
        ends = meta_ref[1:2, 0:_B]
        rows = (i * rb + lax.broadcasted_iota(jnp.int32, (rb, _B), 0)
                ).astype(jnp.float32)
        oneh = jnp.where((rows >= starts) & (rows < ends), 1.0, 0.0)
        o_ref[...] = jnp.dot(s * oneh, jnp.ones((_B, _B), jnp.float32),
                             precision=_HIGH, preferred_element_type=jnp.float32)

    return pl.pallas_call(
        body,
        grid=(grid,),
        in_specs=[
            pl.BlockSpec((rb, _D), lambda i: (i, 0)),
            pl.BlockSpec((_B, _D), lambda i: (0, 0)),
            pl.BlockSpec((8, 128), lambda i: (0, 0)),
        ],
        out_specs=pl.BlockSpec((rb, _B), lambda i: (i, 0)),
        out_shape=jax.ShapeDtypeStruct((_N, _B), jnp.float32),
    )(x, hidden, meta)


def _tc_combine(mt, dt, cn, rh, w_lin, b2):
    def body(mt_ref, dt_ref, cn_ref, rh_ref, wl_ref, b_ref, o_ref):
        rh_v = rh_ref[...]
        valid = rh_v > 0
        mt_v = mt_ref[...]
        mseg = jnp.max(jnp.where(valid, mt_v, _NEG), axis=1, keepdims=True)
        alpha = jnp.exp(jnp.where(valid, mt_v - mseg, _NEG))
        denom = jnp.sum(alpha * dt_ref[...], axis=1, keepdims=True)
        ctx = jnp.dot(alpha, cn_ref[...], precision=_HIGH,
                      preferred_element_type=jnp.float32) / denom
        out = lax.dot_general(ctx, wl_ref[...], (((1,), (1,)), ((), ())),
                              precision=_HIGH,
                              preferred_element_type=jnp.float32)
        o_ref[...] = out + b_ref[0:1, :]

    return pl.pallas_call(
        body, out_shape=jax.ShapeDtypeStruct((_B, _C), jnp.float32),
    )(mt, dt, cn, rh, w_lin, b2)


_k1 = _make_k1()
_k2 = _make_k2()


def kernel(encoded_paths, contexts_per_label, W_attn, W_lin, b_lin):
    x = encoded_paths
    counts = contexts_per_label.astype(jnp.int32)
    off = jnp.cumsum(counts)
    starts = off - counts
    lo = jnp.arange(_NW, dtype=jnp.int32) * _CHUNK
    seg0 = jnp.searchsorted(off, lo, side="right").astype(jnp.int32)
    end0 = jnp.take(off, seg0)
    len0 = jnp.minimum(end0 - lo, _CHUNK)
    seg1 = jnp.minimum(seg0 + 1, _B - 1)
    run_seg = jnp.stack([seg0, seg1], axis=1).reshape(2 * _NW)
    rh = (run_seg[None, :] == jnp.arange(_B, dtype=jnp.int32)[:, None]
          ).astype(jnp.float32)
    r_mean = rh / counts.astype(jnp.float32)[:, None]
    sched = jnp.zeros((_NW, 16), jnp.int32).at[:, 0].set(len0)
    meta = (jnp.zeros((8, 128), jnp.float32)
            .at[0, :_B].set(starts.astype(jnp.float32))
            .at[1, :_B].set(off.astype(jnp.float32)))
    b2 = jnp.broadcast_to(b_lin, (8, _C))

    p = _k1(x, sched)
    hidden = _tc_prep(p.reshape(2 * _NW, _D), r_mean, W_attn)
    s = _tc_score(x, hidden, meta)
    m, d, c = _k2(x, s, sched)
    mt = jnp.transpose(m.reshape(2 * _NW, 16))
    dt = jnp.transpose(d.reshape(2 * _NW, 16))
    return _tc_combine(mt, dt, c.reshape(2 * _NW, _D), rh, W_lin, b2)


# speedup vs baseline: 4.9822x; 1.3293x over previous
import functools

import jax
import jax.numpy as jnp
from jax import lax
from jax.experimental import pallas as pl
from jax.experimental.pallas import tpu as pltpu
from jax.experimental.pallas import tpu_sc as plsc

_N = 32768
_D = 512
_B = 16
_C = 104
_NW = 32
_CHUNK = _N // _NW
_ROWS = 64
_NCH = _CHUNK // _ROWS
_NJ = _D // 16
_NEG = -1.0e30

_HIGH = jax.lax.Precision.HIGHEST


def _wid():
    info = plsc.get_sparse_core_info()
    return lax.axis_index("s") * info.num_cores + lax.axis_index("c")


def _len0_scalar(schedv):
    return schedv[...][0]


def _make_k1():
    mesh = plsc.VectorSubcoreMesh(core_axis_name="c", subcore_axis_name="s")

    @functools.partial(
        pl.kernel,
        mesh=mesh,
        out_type=jax.ShapeDtypeStruct((_NW, 2, _D), jnp.float32),
        scratch_types=[
            pltpu.VMEM((2, _ROWS, _D), jnp.float32),
            pltpu.VMEM((2, _D), jnp.float32),
            pltpu.VMEM((16,), jnp.int32),
            pltpu.SemaphoreType.DMA,
            pltpu.SemaphoreType.DMA,
        ],
    )
    def k1(x_hbm, sched_hbm, out_hbm, xbuf, acc, schedv, sem0, sem1):
        wid = _wid()
        base = wid * _CHUNK
        pltpu.sync_copy(sched_hbm.at[wid], schedv)
        len0 = _len0_scalar(schedv)

        zeros = jnp.zeros((16,), jnp.float32)
        for run in range(2):
            for j in range(_NJ):
                acc[run, pl.ds(16 * j, 16)] = zeros

        sems = (sem0, sem1)

        def dma(c):
            return pltpu.make_async_copy(
                x_hbm.at[pl.ds(base + c * _ROWS, _ROWS), :],
                xbuf.at[c % 2],
                sems[c % 2],
            )

        dma(0).start()
        for c in range(_NCH):
            if c + 1 < _NCH:
                dma(c + 1).start()
            dma(c).wait()
            buf = c % 2
            s_split = jnp.clip(len0 - c * _ROWS, 0, _ROWS)
            for run, lo, hi in ((0, 0, s_split), (1, s_split, _ROWS)):
                init = tuple(acc[run, pl.ds(16 * j, 16)] for j in range(_NJ))

                def body(r, carry, buf=buf):
                    return tuple(
                        carry[j] + xbuf[buf, r, pl.ds(16 * j, 16)]
                        for j in range(_NJ)
                    )

                res = lax.fori_loop(lo, hi, body, init)
                for j in range(_NJ):
                    acc[run, pl.ds(16 * j, 16)] = res[j]

        pltpu.sync_copy(acc, out_hbm.at[wid])

    return k1


def _make_k2():
    mesh = plsc.VectorSubcoreMesh(core_axis_name="c", subcore_axis_name="s")

    @functools.partial(
        pl.kernel,
        mesh=mesh,
        out_type=(
            jax.ShapeDtypeStruct((_NW, 2, 16), jnp.float32),
            jax.ShapeDtypeStruct((_NW, 2, 16), jnp.float32),
            jax.ShapeDtypeStruct((_NW, 2, _D), jnp.float32),
        ),
        scratch_types=[
            pltpu.VMEM((2, _ROWS, _D), jnp.float32),
            pltpu.VMEM((2, _ROWS, 16), jnp.float32),
            pltpu.VMEM((2, _D), jnp.float32),
            pltpu.VMEM((2, 16), jnp.float32),
            pltpu.VMEM((2, 16), jnp.float32),
            pltpu.VMEM((16,), jnp.int32),
            pltpu.SemaphoreType.DMA,
            pltpu.SemaphoreType.DMA,
            pltpu.SemaphoreType.DMA,
            pltpu.SemaphoreType.DMA,
        ],
    )
    def k2(x_hbm, s_hbm, sched_hbm, m_hbm, d_hbm, c_hbm,
           xbuf, svbuf, cacc, dacc, mv, schedv, sem0, sem1, sem2, sem3):
        wid = _wid()
        base = wid * _CHUNK
        pltpu.sync_copy(sched_hbm.at[wid], schedv)
        len0 = _len0_scalar(schedv)

        xsems = (sem0, sem1)
        ssems = (sem2, sem3)

        def xdma(c):
            return pltpu.make_async_copy(
                x_hbm.at[pl.ds(base + c * _ROWS, _ROWS), :],
                xbuf.at[c % 2],
                xsems[c % 2],
            )

        def sdma(c):
            return pltpu.make_async_copy(
                s_hbm.at[pl.ds(base + c * _ROWS, _ROWS), :],
                svbuf.at[c % 2],
                ssems[c % 2],
            )

        neg = jnp.full((16,), _NEG, jnp.float32)
        m_a, m_b = neg, neg
        sdma(0).start()
        for c in range(_NCH):
            if c + 1 < _NCH:
                sdma(c + 1).start()
            sdma(c).wait()
            buf = c % 2
            s_split = jnp.clip(len0 - c * _ROWS, 0, _ROWS)
            m_a = lax.fori_loop(
                0, s_split,
                lambda r, m, buf=buf: jnp.maximum(m, svbuf[buf, r]), m_a)
            m_b = lax.fori_loop(
                s_split, _ROWS,
                lambda r, m, buf=buf: jnp.maximum(m, svbuf[buf, r]), m_b)
        mv[0] = m_a
        mv[1] = m_b

        zeros = jnp.zeros((16,), jnp.float32)
        for run in range(2):
            dacc[run] = zeros
            for j in range(_NJ):
                cacc[run, pl.ds(16 * j, 16)] = zeros

        xdma(0).start()
        sdma(0).start()
        for c in range(_NCH):
            if c + 1 < _NCH:
                xdma(c + 1).start()
                sdma(c + 1).start()
            xdma(c).wait()
            sdma(c).wait()
            buf = c % 2
            s_split = jnp.clip(len0 - c * _ROWS, 0, _ROWS)
            for run, lo, hi, m in ((0, 0, s_split, m_a),
                                   (1, s_split, _ROWS, m_b)):
                init = (dacc[run],) + tuple(
                    cacc[run, pl.ds(16 * j, 16)] for j in range(_NJ))

                def body(r, carry, buf=buf, m=m):
                    d = carry[0]
                    cs = carry[1:]
                    e = jnp.exp(svbuf[buf, r] - m)
                    new_cs = tuple(
                        cs[j] + e * xbuf[buf, r, pl.ds(16 * j, 16)]
                        for j in range(_NJ)
                    )
                    return (d + e,) + new_cs

                res = lax.fori_loop(lo, hi, body, init)
                dacc[run] = res[0]
                for j in range(_NJ):
                    cacc[run, pl.ds(16 * j, 16)] = res[j + 1]

        pltpu.sync_copy(mv, m_hbm.at[wid])
        pltpu.sync_copy(dacc, d_hbm.at[wid])
        pltpu.sync_copy(cacc, c_hbm.at[wid])

    return k2


def _tc_score(x, p64, r_mean, w_attn, meta):
    rb = 2048
    grid = _N // rb

    def body(x_ref, p_ref, r_ref, w_ref, meta_ref, o_ref, h_ref):
        i = pl.program_id(0)

        @pl.when(i == 0)
        def _():
            init = jnp.dot(r_ref[...], p_ref[...], precision=_HIGH,
                           preferred_element_type=jnp.float32)
            h_ref[...] = lax.dot_general(
                init, w_ref[...], (((1,), (1,)), ((), ())), precision=_HIGH,
                preferred_element_type=jnp.float32)

        s = lax.dot_general(x_ref[...], h_ref[...], (((1,), (1,)), ((), ())),
                            precision=jax.lax.Precision.DEFAULT,
                            preferred_element_type=jnp.float32)
        starts = meta_ref[0:1, 0:_B]
        ends = meta_ref[1:2, 0:_B]
        rows = (i * rb + lax.broadcasted_iota(jnp.int32, (rb, _B), 0)
                ).astype(jnp.float32)
        oneh = jnp.where((rows >= starts) & (rows < ends), 1.0, 0.0)
        o_ref[...] = jnp.dot(s * oneh, jnp.ones((_B, _B), jnp.float32),
                             precision=_HIGH, preferred_element_type=jnp.float32)

    return pl.pallas_call(
        body,
        grid=(grid,),
        in_specs=[
            pl.BlockSpec((rb, _D), lambda i: (i, 0)),
            pl.BlockSpec((2 * _NW, _D), lambda i: (0, 0)),
            pl.BlockSpec((_B, 2 * _NW), lambda i: (0, 0)),
            pl.BlockSpec((_D, _D), lambda i: (0, 0)),
            pl.BlockSpec((8, 128), lambda i: (0, 0)),
        ],
        out_specs=pl.BlockSpec((rb, _B), lambda i: (i, 0)),
        out_shape=jax.ShapeDtypeStruct((_N, _B), jnp.float32),
        scratch_shapes=[pltpu.VMEM((_B, _D), jnp.float32)],
    )(x, p64, r_mean, w_attn, meta)


def _tc_combine(mt, dt, cn, rh, w_lin, b2):
    def body(mt_ref, dt_ref, cn_ref, rh_ref, wl_ref, b_ref, o_ref):
        rh_v = rh_ref[...]
        valid = rh_v > 0
        mt_v = mt_ref[...]
        mseg = jnp.max(jnp.where(valid, mt_v, _NEG), axis=1, keepdims=True)
        alpha = jnp.exp(jnp.where(valid, mt_v - mseg, _NEG))
        denom = jnp.sum(alpha * dt_ref[...], axis=1, keepdims=True)
        ctx = jnp.dot(alpha, cn_ref[...], precision=_HIGH,
                      preferred_element_type=jnp.float32) / denom
        out = lax.dot_general(ctx, wl_ref[...], (((1,), (1,)), ((), ())),
                              precision=_HIGH,
                              preferred_element_type=jnp.float32)
        o_ref[...] = out + b_ref[0:1, :]

    return pl.pallas_call(
        body, out_shape=jax.ShapeDtypeStruct((_B, _C), jnp.float32),
    )(mt, dt, cn, rh, w_lin, b2)


_k1 = _make_k1()
_k2 = _make_k2()


def kernel(encoded_paths, contexts_per_label, W_attn, W_lin, b_lin):
    x = encoded_paths
    counts = contexts_per_label.astype(jnp.int32)
    off = jnp.cumsum(counts)
    starts = off - counts
    lo = jnp.arange(_NW, dtype=jnp.int32) * _CHUNK
    seg0 = jnp.searchsorted(off, lo, side="right").astype(jnp.int32)
    end0 = jnp.take(off, seg0)
    len0 = jnp.minimum(end0 - lo, _CHUNK)
    seg1 = jnp.minimum(seg0 + 1, _B - 1)
    run_seg = jnp.stack([seg0, seg1], axis=1).reshape(2 * _NW)
    rh = (run_seg[None, :] == jnp.arange(_B, dtype=jnp.int32)[:, None]
          ).astype(jnp.float32)
    r_mean = rh / counts.astype(jnp.float32)[:, None]
    sched = jnp.zeros((_NW, 16), jnp.int32).at[:, 0].set(len0)
    meta = (jnp.zeros((8, 128), jnp.float32)
            .at[0, :_B].set(starts.astype(jnp.float32))
            .at[1, :_B].set(off.astype(jnp.float32)))
    b2 = jnp.broadcast_to(b_lin, (8, _C))

    p = _k1(x, sched)
    s = _tc_score(x, p.reshape(2 * _NW, _D), r_mean, W_attn, meta)
    m, d, c = _k2(x, s, sched)
    mt = jnp.transpose(m.reshape(2 * _NW, 16))
    dt = jnp.transpose(d.reshape(2 * _NW, 16))
    return _tc_combine(mt, dt, c.reshape(2 * _NW, _D), rh, W_lin, b2)


# speedup vs baseline: 5.7380x; 1.1517x over previous
import functools

import jax
import jax.numpy as jnp
from jax import lax
from jax.experimental import pallas as pl
from jax.experimental.pallas import tpu as pltpu
from jax.experimental.pallas import tpu_sc as plsc

_N = 32768
_D = 512
_B = 16
_C = 104
_NW = 32
_NR = 2 * _NW
_CHUNK = _N // _NW
_ROWS = 64
_NCH = _CHUNK // _ROWS
_NJ = _D // 16
_RB = 2048
_NG = _N // _RB
_NEG = -1.0e30

_HIGH = jax.lax.Precision.HIGHEST


def _wid():
    info = plsc.get_sparse_core_info()
    return lax.axis_index("s") * info.num_cores + lax.axis_index("c")


def _len0_scalar(schedv):
    return schedv[...][0]


def _make_k1():
    mesh = plsc.VectorSubcoreMesh(core_axis_name="c", subcore_axis_name="s")

    @functools.partial(
        pl.kernel,
        mesh=mesh,
        out_type=jax.ShapeDtypeStruct((_NR, _D), jnp.float32),
        scratch_types=[
            pltpu.VMEM((2, _ROWS, _D), jnp.float32),
            pltpu.VMEM((2, _D), jnp.float32),
            pltpu.VMEM((16,), jnp.int32),
            pltpu.SemaphoreType.DMA,
            pltpu.SemaphoreType.DMA,
        ],
    )
    def k1(x_hbm, sched_hbm, out_hbm, xbuf, acc, schedv, sem0, sem1):
        wid = _wid()
        base = wid * _CHUNK
        pltpu.sync_copy(sched_hbm.at[wid], schedv)
        len0 = _len0_scalar(schedv)

        zeros = jnp.zeros((16,), jnp.float32)
        for run in range(2):
            for j in range(_NJ):
                acc[run, pl.ds(16 * j, 16)] = zeros

        sems = (sem0, sem1)

        def dma(c):
            return pltpu.make_async_copy(
                x_hbm.at[pl.ds(base + c * _ROWS, _ROWS), :],
                xbuf.at[c % 2],
                sems[c % 2],
            )

        dma(0).start()
        for c in range(_NCH):
            if c + 1 < _NCH:
                dma(c + 1).start()
            dma(c).wait()
            buf = c % 2
            s_split = jnp.clip(len0 - c * _ROWS, 0, _ROWS)
            for run, lo, hi in ((0, 0, s_split), (1, s_split, _ROWS)):
                init = tuple(acc[run, pl.ds(16 * j, 16)] for j in range(_NJ))

                def body(r, carry, buf=buf):
                    return tuple(
                        carry[j] + xbuf[buf, r, pl.ds(16 * j, 16)]
                        for j in range(_NJ)
                    )

                res = lax.fori_loop(lo, hi, body, init)
                for j in range(_NJ):
                    acc[run, pl.ds(16 * j, 16)] = res[j]

        pltpu.sync_copy(acc, out_hbm.at[pl.ds(2 * wid, 2), :])

    return k1


def _make_k2():
    mesh = plsc.VectorSubcoreMesh(core_axis_name="c", subcore_axis_name="s")

    @functools.partial(
        pl.kernel,
        mesh=mesh,
        out_type=jax.ShapeDtypeStruct((_NR, _D), jnp.float32),
        scratch_types=[
            pltpu.VMEM((2, _ROWS, _D), jnp.float32),
            pltpu.VMEM((2, _ROWS, 16), jnp.float32),
            pltpu.VMEM((2, _D), jnp.float32),
            pltpu.VMEM((16,), jnp.int32),
            pltpu.SemaphoreType.DMA,
            pltpu.SemaphoreType.DMA,
            pltpu.SemaphoreType.DMA,
            pltpu.SemaphoreType.DMA,
        ],
    )
    def k2(x_hbm, e_hbm, sched_hbm, c_hbm,
           xbuf, ebuf, cacc, schedv, sem0, sem1, sem2, sem3):
        wid = _wid()
        base = wid * _CHUNK
        pltpu.sync_copy(sched_hbm.at[wid], schedv)
        len0 = _len0_scalar(schedv)

        xsems = (sem0, sem1)
        esems = (sem2, sem3)

        def xdma(c):
            return pltpu.make_async_copy(
                x_hbm.at[pl.ds(base + c * _ROWS, _ROWS), :],
                xbuf.at[c % 2],
                xsems[c % 2],
            )

        def edma(c):
            return pltpu.make_async_copy(
                e_hbm.at[pl.ds(base + c * _ROWS, _ROWS), :],
                ebuf.at[c % 2],
                esems[c % 2],
            )

        zeros = jnp.zeros((16,), jnp.float32)
        for run in range(2):
            for j in range(_NJ):
                cacc[run, pl.ds(16 * j, 16)] = zeros

        xdma(0).start()
        edma(0).start()
        for c in range(_NCH):
            if c + 1 < _NCH:
                xdma(c + 1).start()
                edma(c + 1).start()
            xdma(c).wait()
            edma(c).wait()
            buf = c % 2
            s_split = jnp.clip(len0 - c * _ROWS, 0, _ROWS)
            for run, lo, hi in ((0, 0, s_split), (1, s_split, _ROWS)):
                init = tuple(cacc[run, pl.ds(16 * j, 16)] for j in range(_NJ))

                def body(r, carry, buf=buf):
                    e = ebuf[buf, r]
                    return tuple(
                        carry[j] + e * xbuf[buf, r, pl.ds(16 * j, 16)]
                        for j in range(_NJ)
                    )

                res = lax.fori_loop(lo, hi, body, init)
                for j in range(_NJ):
                    cacc[run, pl.ds(16 * j, 16)] = res[j]

        pltpu.sync_copy(cacc, c_hbm.at[pl.ds(2 * wid, 2), :])

    return k2


def _tc_score(x, p, r_mean, w_attn, meta):
    def body(x_ref, p_ref, r_ref, w_ref, meta_ref,
             e_ref, mo_ref, do_ref, h_ref):
        i = pl.program_id(0)

        @pl.when(i == 0)
        def _():
            init = jnp.dot(r_ref[...], p_ref[...], precision=_HIGH,
                           preferred_element_type=jnp.float32)
            h_ref[...] = lax.dot_general(
                init, w_ref[...], (((1,), (1,)), ((), ())), precision=_HIGH,
                preferred_element_type=jnp.float32)

        s_all = lax.dot_general(
            x_ref[...], h_ref[...], (((1,), (1,)), ((), ())),
            precision=jax.lax.Precision.DEFAULT,
            preferred_element_type=jnp.float32)
        rows_b = (i * _RB + lax.broadcasted_iota(jnp.int32, (_RB, _B), 0)
                  ).astype(jnp.float32)
        oneh_seg = jnp.where(
            (rows_b >= meta_ref[0:1, 0:_B]) & (rows_b < meta_ref[1:2, 0:_B]),
            1.0, 0.0)
        s = jnp.sum(s_all * oneh_seg, axis=1, keepdims=True)

        rows_r = (i * _RB + lax.broadcasted_iota(jnp.int32, (_RB, _NR), 0)
                  ).astype(jnp.float32)
        oneh_run = jnp.where(
            (rows_r >= meta_ref[2:3, 0:_NR]) & (rows_r < meta_ref[3:4, 0:_NR]),
            1.0, 0.0)
        m_run = jnp.max(jnp.where(oneh_run > 0, s, _NEG),
                        axis=0, keepdims=True)
        m_row = jnp.dot(oneh_run, jnp.transpose(m_run), precision=_HIGH,
                        preferred_element_type=jnp.float32)
        e = jnp.exp(s - m_row)
        e_ref[...] = jnp.broadcast_to(e, (_RB, _B))
        d_run = lax.dot_general(e, oneh_run, (((0,), (0,)), ((), ())),
                                precision=_HIGH,
                                preferred_element_type=jnp.float32)
        srow = lax.broadcasted_iota(jnp.int32, (8, _NR), 0)
        mo_ref[...] = jnp.where(srow == 0, jnp.broadcast_to(m_run, (8, _NR)),
                                _NEG)
        do_ref[...] = jnp.where(srow == 0, jnp.broadcast_to(d_run, (8, _NR)),
                                0.0)

    return pl.pallas_call(
        body,
        grid=(_NG,),
        in_specs=[
            pl.BlockSpec((_RB, _D), lambda i: (i, 0)),
            pl.BlockSpec((_NR, _D), lambda i: (0, 0)),
            pl.BlockSpec((_B, _NR), lambda i: (0, 0)),
            pl.BlockSpec((_D, _D), lambda i: (0, 0)),
            pl.BlockSpec((8, 128), lambda i: (0, 0)),
        ],
        out_specs=(
            pl.BlockSpec((_RB, _B), lambda i: (i, 0)),
            pl.BlockSpec((8, _NR), lambda i: (i, 0)),
            pl.BlockSpec((8, _NR), lambda i: (i, 0)),
        ),
        out_shape=(
            jax.ShapeDtypeStruct((_N, _B), jnp.float32),
            jax.ShapeDtypeStruct((8 * _NG, _NR), jnp.float32),
            jax.ShapeDtypeStruct((8 * _NG, _NR), jnp.float32),
        ),
        scratch_shapes=[pltpu.VMEM((_B, _D), jnp.float32)],
    )(x, p, r_mean, w_attn, meta)


def _tc_combine(mo, do, cn, rh, w_lin, b2):
    def body(mo_ref, do_ref, cn_ref, rh_ref, wl_ref, b_ref, o_ref):
        m_run = jnp.max(mo_ref[...], axis=0, keepdims=True)
        d_run = jnp.sum(do_ref[...], axis=0, keepdims=True)
        rh_v = rh_ref[...]
        valid = rh_v > 0
        mt = jnp.broadcast_to(m_run, (_B, _NR))
        mseg = jnp.max(jnp.where(valid, mt, _NEG), axis=1, keepdims=True)
        alpha = jnp.exp(jnp.where(valid, mt - mseg, _NEG))
        denom = jnp.sum(alpha * d_run, axis=1, keepdims=True)
        ctx = jnp.dot(alpha, cn_ref[...], precision=_HIGH,
                      preferred_element_type=jnp.float32) / denom
        out = lax.dot_general(ctx, wl_ref[...], (((1,), (1,)), ((), ())),
                              precision=_HIGH,
                              preferred_element_type=jnp.float32)
        o_ref[...] = out + b_ref[0:1, :]

    return pl.pallas_call(
        body, out_shape=jax.ShapeDtypeStruct((_B, _C), jnp.float32),
    )(mo, do, cn, rh, w_lin, b2)


_k1 = _make_k1()
_k2 = _make_k2()


def kernel(encoded_paths, contexts_per_label, W_attn, W_lin, b_lin):
    x = encoded_paths
    counts = contexts_per_label.astype(jnp.int32)
    off = jnp.cumsum(counts)
    starts = off - counts
    lo = jnp.arange(_NW, dtype=jnp.int32) * _CHUNK
    seg0 = jnp.searchsorted(off, lo, side="right").astype(jnp.int32)
    end0 = jnp.take(off, seg0)
    len0 = jnp.minimum(end0 - lo, _CHUNK)
    seg1 = jnp.minimum(seg0 + 1, _B - 1)
    run_seg = jnp.stack([seg0, seg1], axis=1).reshape(_NR)
    rh = (run_seg[None, :] == jnp.arange(_B, dtype=jnp.int32)[:, None]
          ).astype(jnp.float32)
    r_mean = rh / counts.astype(jnp.float32)[:, None]
    sched = jnp.zeros((_NW, 16), jnp.int32).at[:, 0].set(len0)
    split = lo + len0
    run_start = jnp.stack([lo, split], axis=1).reshape(_NR)
    run_end = jnp.stack([split, lo + _CHUNK], axis=1).reshape(_NR)
    meta = (jnp.zeros((8, 128), jnp.float32)
            .at[0, :_B].set(starts.astype(jnp.float32))
            .at[1, :_B].set(off.astype(jnp.float32))
            .at[2, :_NR].set(run_start.astype(jnp.float32))
            .at[3, :_NR].set(run_end.astype(jnp.float32)))
    b2 = jnp.broadcast_to(b_lin, (8, _C))

    p = _k1(x, sched)
    e, mo, do = _tc_score(x, p, r_mean, W_attn, meta)
    c = _k2(x, e, sched)
    return _tc_combine(mo, do, c, rh, W_lin, b2)


# speedup vs baseline: 6.4743x; 1.1283x over previous
import functools

import jax
import jax.numpy as jnp
from jax import lax
from jax.experimental import pallas as pl
from jax.experimental.pallas import tpu as pltpu
from jax.experimental.pallas import tpu_sc as plsc

_N = 32768
_D = 512
_B = 16
_C = 104
_NW = 32
_NR = 2 * _NW
_CHUNK = _N // _NW
_ROWS = 64
_NCH = _CHUNK // _ROWS
_NJ = _D // 16
_NEG = -1.0e30

_HIGH = jax.lax.Precision.HIGHEST


def _wid():
    info = plsc.get_sparse_core_info()
    return lax.axis_index("s") * info.num_cores + lax.axis_index("c")


def _hsum16(v):
    iota = lax.iota(jnp.int32, 16)
    for st in (8, 4, 2, 1):
        idx = jnp.bitwise_xor(iota, st)
        v = v + v.at[idx].get(mode="promise_in_bounds", unique_indices=True)
    return v


def _make_k1():
    mesh = plsc.VectorSubcoreMesh(core_axis_name="c", subcore_axis_name="s")

    @functools.partial(
        pl.kernel,
        mesh=mesh,
        out_type=jax.ShapeDtypeStruct((_NR, _D), jnp.float32),
        scratch_types=[
            pltpu.VMEM((2, _ROWS, _D), jnp.float32),
            pltpu.VMEM((2, _D), jnp.float32),
            pltpu.VMEM((16,), jnp.int32),
            pltpu.SemaphoreType.DMA,
            pltpu.SemaphoreType.DMA,
        ],
    )
    def k1(x_hbm, sched_hbm, out_hbm, xbuf, acc, schedv, sem0, sem1):
        wid = _wid()
        base = wid * _CHUNK
        pltpu.sync_copy(sched_hbm.at[wid], schedv)
        len0 = schedv[...][0]

        zeros = jnp.zeros((16,), jnp.float32)
        for run in range(2):
            for j in range(_NJ):
                acc[run, pl.ds(16 * j, 16)] = zeros

        sems = (sem0, sem1)

        def dma(c):
            return pltpu.make_async_copy(
                x_hbm.at[pl.ds(base + c * _ROWS, _ROWS), :],
                xbuf.at[c % 2],
                sems[c % 2],
            )

        dma(0).start()
        for c in range(_NCH):
            if c + 1 < _NCH:
                dma(c + 1).start()
            dma(c).wait()
            buf = c % 2
            s_split = jnp.clip(len0 - c * _ROWS, 0, _ROWS)
            for run, lo, hi in ((0, 0, s_split), (1, s_split, _ROWS)):
                init = tuple(acc[run, pl.ds(16 * j, 16)] for j in range(_NJ))

                def body(r, carry, buf=buf):
                    return tuple(
                        carry[j] + xbuf[buf, r, pl.ds(16 * j, 16)]
                        for j in range(_NJ)
                    )

                res = lax.fori_loop(lo, hi, body, init)
                for j in range(_NJ):
                    acc[run, pl.ds(16 * j, 16)] = res[j]

        pltpu.sync_copy(acc, out_hbm.at[pl.ds(2 * wid, 2), :])

    return k1


def _make_k2():
    mesh = plsc.VectorSubcoreMesh(core_axis_name="c", subcore_axis_name="s")

    @functools.partial(
        pl.kernel,
        mesh=mesh,
        out_type=(
            jax.ShapeDtypeStruct((_NR, 16), jnp.float32),
            jax.ShapeDtypeStruct((_NR, 16), jnp.float32),
            jax.ShapeDtypeStruct((_NR, _D), jnp.float32),
        ),
        scratch_types=[
            pltpu.VMEM((2, _ROWS, _D), jnp.float32),
            pltpu.VMEM((_ROWS, 16), jnp.float32),
            pltpu.VMEM((_B, _D), jnp.float32),
            pltpu.VMEM((2, _D), jnp.float32),
            pltpu.VMEM((2, 16), jnp.float32),
            pltpu.VMEM((2, 16), jnp.float32),
            pltpu.VMEM((16,), jnp.int32),
            pltpu.SemaphoreType.DMA,
            pltpu.SemaphoreType.DMA,
            pltpu.SemaphoreType.DMA,
        ],
    )
    def k2(x_hbm, h_hbm, sched_hbm, m_hbm, d_hbm, c_hbm,
           xbuf, sbuf, hbuf, cacc, dacc, mvv, schedv, sem0, sem1, semh):
        wid = _wid()
        base = wid * _CHUNK
        pltpu.sync_copy(sched_hbm.at[wid], schedv)
        len0 = schedv[...][0]
        seg0 = schedv[...][1]
        pltpu.make_async_copy(h_hbm, hbuf, semh).start()

        neg = jnp.full((16,), _NEG, jnp.float32)
        zeros = jnp.zeros((16,), jnp.float32)
        for run in range(2):
            mvv[run] = neg
            dacc[run] = zeros
            for j in range(_NJ):
                cacc[run, pl.ds(16 * j, 16)] = zeros

        sems = (sem0, sem1)

        def xdma(c, buf):
            return pltpu.make_async_copy(
                x_hbm.at[pl.ds(base + c * _ROWS, _ROWS), :],
                xbuf.at[buf],
                sems[buf],
            )

        def process(c, buf):
            s_split = jnp.clip(len0 - c * _ROWS, 0, _ROWS)
            for runi, lo, hi, seg in ((0, 0, s_split, seg0),
                                      (1, s_split, _ROWS, seg0 + 1)):
                segc = jnp.minimum(seg, _B - 1)
                hs = tuple(hbuf[segc, pl.ds(16 * j, 16)] for j in range(_NJ))

                def dbody(r, smax, hs=hs, buf=buf):
                    parts = [xbuf[buf, r, pl.ds(16 * j, 16)] * hs[j]
                             for j in range(_NJ)]
                    while len(parts) > 1:
                        nxt = [parts[i] + parts[i + 1]
                               for i in range(0, len(parts) - 1, 2)]
                        if len(parts) % 2:
                            nxt.append(parts[-1])
                        parts = nxt
                    svec = _hsum16(parts[0])
                    sbuf[r] = svec
                    return jnp.maximum(smax, svec)

                smax = lax.fori_loop(lo, hi, dbody, neg)

                m_old = mvv[runi]
                m_new = jnp.maximum(m_old, smax)
                gam = jnp.exp(m_old - m_new)
                mvv[runi] = m_new
                init = (dacc[runi] * gam,) + tuple(
                    cacc[runi, pl.ds(16 * j, 16)] * gam for j in range(_NJ))

                def wbody(r, carry, buf=buf, m_new=m_new):
                    d = carry[0]
                    cs = carry[1:]
                    w = jnp.exp(sbuf[r] - m_new)
                    ncs = tuple(
                        cs[j] + w * xbuf[buf, r, pl.ds(16 * j, 16)]
                        for j in range(_NJ)
                    )
                    return (d + w,) + ncs

                res = lax.fori_loop(lo, hi, wbody, init)
                dacc[runi] = res[0]
                for j in range(_NJ):
                    cacc[runi, pl.ds(16 * j, 16)] = res[j + 1]

        xdma(0, 0).start()
        pltpu.make_async_copy(h_hbm, hbuf, semh).wait()

        def gbody(g, carry):
            c0 = 2 * g
            xdma(c0 + 1, 1).start()
            xdma(c0, 0).wait()
            process(c0, 0)

            @pl.when(c0 + 2 < _NCH)
            def _():
                xdma(c0 + 2, 0).start()

            xdma(c0 + 1, 1).wait()
            process(c0 + 1, 1)
            return carry

        lax.fori_loop(0, _NCH // 2, gbody, 0)

        pltpu.sync_copy(mvv, m_hbm.at[pl.ds(2 * wid, 2), :])
        pltpu.sync_copy(dacc, d_hbm.at[pl.ds(2 * wid, 2), :])
        pltpu.sync_copy(cacc, c_hbm.at[pl.ds(2 * wid, 2), :])

    return k2


def _tc_prep(p, r_mean, w_attn):
    def body(p_ref, r_ref, w_ref, o_ref):
        init = jnp.dot(r_ref[...], p_ref[...], precision=_HIGH,
                       preferred_element_type=jnp.float32)
        o_ref[...] = lax.dot_general(
            init, w_ref[...], (((1,), (1,)), ((), ())), precision=_HIGH,
            preferred_element_type=jnp.float32)

    return pl.pallas_call(
        body, out_shape=jax.ShapeDtypeStruct((_B, _D), jnp.float32),
    )(p, r_mean, w_attn)


def _tc_combine(m, d, cn, rh, w_lin, b2):
    def body(m_ref, d_ref, cn_ref, rh_ref, wl_ref, b_ref, o_ref):
        ones_b = jnp.ones((_B, 1), jnp.float32)
        mt = lax.dot_general(ones_b, m_ref[:, 0:1], (((1,), (1,)), ((), ())),
                             precision=_HIGH,
                             preferred_element_type=jnp.float32)
        dt = lax.dot_general(ones_b, d_ref[:, 0:1], (((1,), (1,)), ((), ())),
                             precision=_HIGH,
                             preferred_element_type=jnp.float32)
        rh_v = rh_ref[...]
        valid = rh_v > 0
        mseg = jnp.max(jnp.where(valid, mt, _NEG), axis=1, keepdims=True)
        alpha = jnp.exp(jnp.where(valid, mt - mseg, _NEG))
        denom = jnp.sum(alpha * dt, axis=1, keepdims=True)
        ctx = jnp.dot(alpha, cn_ref[...], precision=_HIGH,
                      preferred_element_type=jnp.float32) / denom
        out = lax.dot_general(ctx, wl_ref[...], (((1,), (1,)), ((), ())),
                              precision=_HIGH,
                              preferred_element_type=jnp.float32)
        o_ref[...] = out + b_ref[0:1, :]

    return pl.pallas_call(
        body, out_shape=jax.ShapeDtypeStruct((_B, _C), jnp.float32),
    )(m, d, cn, rh, w_lin, b2)


_k1 = _make_k1()
_k2 = _make_k2()


def kernel(encoded_paths, contexts_per_label, W_attn, W_lin, b_lin):
    x = encoded_paths
    counts = contexts_per_label.astype(jnp.int32)
    off = jnp.cumsum(counts)
    lo = jnp.arange(_NW, dtype=jnp.int32) * _CHUNK
    seg0 = jnp.searchsorted(off, lo, side="right").astype(jnp.int32)
    end0 = jnp.take(off, seg0)
    len0 = jnp.minimum(end0 - lo, _CHUNK)
    seg1 = jnp.minimum(seg0 + 1, _B - 1)
    run_seg = jnp.stack([seg0, seg1], axis=1).reshape(_NR)
    rh = (run_seg[None, :] == jnp.arange(_B, dtype=jnp.int32)[:, None]
          ).astype(jnp.float32)
    r_mean = rh / counts.astype(jnp.float32)[:, None]
    sched = (jnp.zeros((_NW, 16), jnp.int32)
             .at[:, 0].set(len0)
             .at[:, 1].set(seg0))
    b2 = jnp.broadcast_to(b_lin, (8, _C))

    p = _k1(x, sched)
    hidden = _tc_prep(p, r_mean, W_attn)
    m, d, c = _k2(x, hidden, sched)
    return _tc_combine(m, d, c, rh, W_lin, b2)


# speedup vs baseline: 6.7357x; 1.0404x over previous
import functools

import jax
import jax.numpy as jnp
from jax import lax
from jax.experimental import pallas as pl
from jax.experimental.pallas import tpu as pltpu
from jax.experimental.pallas import tpu_sc as plsc

_N = 32768
_D = 512
_B = 16
_C = 104
_NW = 32
_NR = 2 * _NW
_CHUNK = _N // _NW
_ROWS = 64
_NCH = _CHUNK // _ROWS
_NJ = _D // 16
_NEG = -1.0e30

_HIGH = jax.lax.Precision.HIGHEST


def _wid():
    info = plsc.get_sparse_core_info()
    return lax.axis_index("s") * info.num_cores + lax.axis_index("c")


def _hsum16(v):
    iota = lax.iota(jnp.int32, 16)
    for st in (8, 4, 2, 1):
        idx = jnp.bitwise_xor(iota, st)
        v = v + v.at[idx].get(mode="promise_in_bounds", unique_indices=True)
    return v


def _make_k1():
    mesh = plsc.VectorSubcoreMesh(core_axis_name="c", subcore_axis_name="s")

    @functools.partial(
        pl.kernel,
        mesh=mesh,
        out_type=jax.ShapeDtypeStruct((_NR, _D), jnp.float32),
        scratch_types=[
            pltpu.VMEM((2, _ROWS, _D), jnp.float32),
            pltpu.VMEM((2, _D), jnp.float32),
            pltpu.VMEM((16,), jnp.int32),
            pltpu.SemaphoreType.DMA,
            pltpu.SemaphoreType.DMA,
        ],
    )
    def k1(x_hbm, sched_hbm, out_hbm, xbuf, acc, schedv, sem0, sem1):
        wid = _wid()
        base = wid * _CHUNK
        pltpu.sync_copy(sched_hbm.at[wid], schedv)
        len0 = schedv[...][0]

        zeros = jnp.zeros((16,), jnp.float32)
        for run in range(2):
            for j in range(_NJ):
                acc[run, pl.ds(16 * j, 16)] = zeros

        sems = (sem0, sem1)

        def dma(c, buf):
            return pltpu.make_async_copy(
                x_hbm.at[pl.ds(base + c * _ROWS, _ROWS), :],
                xbuf.at[buf],
                sems[buf],
            )

        def process(c, buf):
            s_split = jnp.clip(len0 - c * _ROWS, 0, _ROWS)
            for run, lo, hi in ((0, 0, s_split), (1, s_split, _ROWS)):
                init = tuple(acc[run, pl.ds(16 * j, 16)] for j in range(_NJ))

                def body(r, carry, buf=buf):
                    return tuple(
                        carry[j] + xbuf[buf, r, pl.ds(16 * j, 16)]
                        for j in range(_NJ)
                    )

                res = plsc.parallel_loop(lo, hi, unroll=4, carry=init)(body)
                for j in range(_NJ):
                    acc[run, pl.ds(16 * j, 16)] = res[j]

        dma(0, 0).start()

        def gbody(g, carry):
            c0 = 2 * g
            dma(c0 + 1, 1).start()
            dma(c0, 0).wait()
            process(c0, 0)

            @pl.when(c0 + 2 < _NCH)
            def _():
                dma(c0 + 2, 0).start()

            dma(c0 + 1, 1).wait()
            process(c0 + 1, 1)
            return carry

        lax.fori_loop(0, _NCH // 2, gbody, 0)

        pltpu.sync_copy(acc, out_hbm.at[pl.ds(2 * wid, 2), :])

    return k1


def _make_k2():
    mesh = plsc.VectorSubcoreMesh(core_axis_name="c", subcore_axis_name="s")

    @functools.partial(
        pl.kernel,
        mesh=mesh,
        out_type=(
            jax.ShapeDtypeStruct((_NR, 16), jnp.float32),
            jax.ShapeDtypeStruct((_NR, 16), jnp.float32),
            jax.ShapeDtypeStruct((_NR, _D), jnp.float32),
        ),
        scratch_types=[
            pltpu.VMEM((2, _ROWS, _D), jnp.float32),
            pltpu.VMEM((_ROWS, 16), jnp.float32),
            pltpu.VMEM((_B, _D), jnp.float32),
            pltpu.VMEM((2, _D), jnp.float32),
            pltpu.VMEM((2, 16), jnp.float32),
            pltpu.VMEM((2, 16), jnp.float32),
            pltpu.VMEM((16,), jnp.int32),
            pltpu.SemaphoreType.DMA,
            pltpu.SemaphoreType.DMA,
            pltpu.SemaphoreType.DMA,
        ],
    )
    def k2(x_hbm, h_hbm, sched_hbm, m_hbm, d_hbm, c_hbm,
           xbuf, sbuf, hbuf, cacc, dacc, mvv, schedv, sem0, sem1, semh):
        wid = _wid()
        base = wid * _CHUNK
        pltpu.sync_copy(sched_hbm.at[wid], schedv)
        len0 = schedv[...][0]
        seg0 = schedv[...][1]
        pltpu.make_async_copy(h_hbm, hbuf, semh).start()

        neg = jnp.full((16,), _NEG, jnp.float32)
        zeros = jnp.zeros((16,), jnp.float32)
        for run in range(2):
            mvv[run] = neg
            dacc[run] = zeros
            for j in range(_NJ):
                cacc[run, pl.ds(16 * j, 16)] = zeros

        sems = (sem0, sem1)

        def xdma(c, buf):
            return pltpu.make_async_copy(
                x_hbm.at[pl.ds(base + c * _ROWS, _ROWS), :],
                xbuf.at[buf],
                sems[buf],
            )

        def process(c, buf):
            s_split = jnp.clip(len0 - c * _ROWS, 0, _ROWS)
            for runi, lo, hi, seg in ((0, 0, s_split, seg0),
                                      (1, s_split, _ROWS, seg0 + 1)):
                segc = jnp.minimum(seg, _B - 1)
                hs = tuple(hbuf[segc, pl.ds(16 * j, 16)] for j in range(_NJ))

                def dbody(r, smax, hs=hs, buf=buf):
                    parts = [xbuf[buf, r, pl.ds(16 * j, 16)] * hs[j]
                             for j in range(_NJ)]
                    while len(parts) > 1:
                        nxt = [parts[i] + parts[i + 1]
                               for i in range(0, len(parts) - 1, 2)]
                        if len(parts) % 2:
                            nxt.append(parts[-1])
                        parts = nxt
                    svec = _hsum16(parts[0])
                    sbuf[r] = svec
                    return jnp.maximum(smax, svec)

                smax = plsc.parallel_loop(lo, hi, unroll=2, carry=neg)(dbody)

                m_old = mvv[runi]
                m_new = jnp.maximum(m_old, smax)
                gam = jnp.exp(m_old - m_new)
                mvv[runi] = m_new
                init = (dacc[runi] * gam,) + tuple(
                    cacc[runi, pl.ds(16 * j, 16)] * gam for j in range(_NJ))

                def wbody(r, carry, buf=buf, m_new=m_new):
                    d = carry[0]
                    cs = carry[1:]
                    w = jnp.exp(sbuf[r] - m_new)
                    ncs = tuple(
                        cs[j] + w * xbuf[buf, r, pl.ds(16 * j, 16)]
                        for j in range(_NJ)
                    )
                    return (d + w,) + ncs

                res = plsc.parallel_loop(lo, hi, unroll=2, carry=init)(wbody)
                dacc[runi] = res[0]
                for j in range(_NJ):
                    cacc[runi, pl.ds(16 * j, 16)] = res[j + 1]

        xdma(0, 0).start()
        pltpu.make_async_copy(h_hbm, hbuf, semh).wait()

        def gbody(g, carry):
            c0 = 2 * g
            xdma(c0 + 1, 1).start()
            xdma(c0, 0).wait()
            process(c0, 0)

            @pl.when(c0 + 2 < _NCH)
            def _():
                xdma(c0 + 2, 0).start()

            xdma(c0 + 1, 1).wait()
            process(c0 + 1, 1)
            return carry

        lax.fori_loop(0, _NCH // 2, gbody, 0)

        pltpu.sync_copy(mvv, m_hbm.at[pl.ds(2 * wid, 2), :])
        pltpu.sync_copy(dacc, d_hbm.at[pl.ds(2 * wid, 2), :])
        pltpu.sync_copy(cacc, c_hbm.at[pl.ds(2 * wid, 2), :])

    return k2


def _tc_prep(p, r_mean, w_attn):
    def body(p_ref, r_ref, w_ref, o_ref):
        init = jnp.dot(r_ref[...], p_ref[...], precision=_HIGH,
                       preferred_element_type=jnp.float32)
        o_ref[...] = lax.dot_general(
            init, w_ref[...], (((1,), (1,)), ((), ())), precision=_HIGH,
            preferred_element_type=jnp.float32)

    return pl.pallas_call(
        body, out_shape=jax.ShapeDtypeStruct((_B, _D), jnp.float32),
    )(p, r_mean, w_attn)


def _tc_combine(m, d, cn, rh, w_lin, b2):
    def body(m_ref, d_ref, cn_ref, rh_ref, wl_ref, b_ref, o_ref):
        ones_b = jnp.ones((_B, 1), jnp.float32)
        mt = lax.dot_general(ones_b, m_ref[:, 0:1], (((1,), (1,)), ((), ())),
                             precision=_HIGH,
                             preferred_element_type=jnp.float32)
        dt = lax.dot_general(ones_b, d_ref[:, 0:1], (((1,), (1,)), ((), ())),
                             precision=_HIGH,
                             preferred_element_type=jnp.float32)
        rh_v = rh_ref[...]
        valid = rh_v > 0
        mseg = jnp.max(jnp.where(valid, mt, _NEG), axis=1, keepdims=True)
        alpha = jnp.exp(jnp.where(valid, mt - mseg, _NEG))
        denom = jnp.sum(alpha * dt, axis=1, keepdims=True)
        ctx = jnp.dot(alpha, cn_ref[...], precision=_HIGH,
                      preferred_element_type=jnp.float32) / denom
        out = lax.dot_general(ctx, wl_ref[...], (((1,), (1,)), ((), ())),
                              precision=_HIGH,
                              preferred_element_type=jnp.float32)
        o_ref[...] = out + b_ref[0:1, :]

    return pl.pallas_call(
        body, out_shape=jax.ShapeDtypeStruct((_B, _C), jnp.float32),
    )(m, d, cn, rh, w_lin, b2)


_k1 = _make_k1()
_k2 = _make_k2()


def kernel(encoded_paths, contexts_per_label, W_attn, W_lin, b_lin):
    x = encoded_paths
    counts = contexts_per_label.astype(jnp.int32)
    off = jnp.cumsum(counts)
    lo = jnp.arange(_NW, dtype=jnp.int32) * _CHUNK
    seg0 = jnp.searchsorted(off, lo, side="right").astype(jnp.int32)
    end0 = jnp.take(off, seg0)
    len0 = jnp.minimum(end0 - lo, _CHUNK)
    seg1 = jnp.minimum(seg0 + 1, _B - 1)
    run_seg = jnp.stack([seg0, seg1], axis=1).reshape(_NR)
    rh = (run_seg[None, :] == jnp.arange(_B, dtype=jnp.int32)[:, None]
          ).astype(jnp.float32)
    r_mean = rh / counts.astype(jnp.float32)[:, None]
    sched = (jnp.zeros((_NW, 16), jnp.int32)
             .at[:, 0].set(len0)
             .at[:, 1].set(seg0))
    b2 = jnp.broadcast_to(b_lin, (8, _C))

    p = _k1(x, sched)
    hidden = _tc_prep(p, r_mean, W_attn)
    m, d, c = _k2(x, hidden, sched)
    return _tc_combine(m, d, c, rh, W_lin, b2)
